# baseline (speedup 1.0000x reference)
import functools

import jax
import jax.numpy as jnp
from jax import lax
from jax.experimental import pallas as pl
from jax.experimental.pallas import tpu as pltpu
from jax.experimental.pallas import tpu_sc as plsc

B, S, D = 1, 2048, 768
H, KVH, DH = 12, 4, 64
E, TOPK, F = 8, 2, 1024
EPS = 1e-6
HALF = DH // 2
REP = H // KVH

TS = 256
TM = 256
NB = (TOPK * S + E * (TM - 1) + TM - 1) // TM
P_TOT = NB * TM
FB = 512
NF = F // FB

NW = 32
CPW = S // NW
CSUB = 32
LANES = 16


def _k1_body(x_ref, s1_ref, wq_ref, wk_ref, wv_ref, cos_ref, sin_ref,
             q_ref, k_ref, v_ref):
    xb = x_ref[...]
    var = jnp.mean(xb * xb, axis=1, keepdims=True)
    h = (xb * lax.rsqrt(var + EPS) * s1_ref[...]).astype(jnp.bfloat16)
    q = jnp.dot(h, wq_ref[...].astype(jnp.bfloat16),
                preferred_element_type=jnp.float32)
    k = jnp.dot(h, wk_ref[...].astype(jnp.bfloat16),
                preferred_element_type=jnp.float32)
    v = jnp.dot(h, wv_ref[...].astype(jnp.bfloat16),
                preferred_element_type=jnp.float32)
    cos = cos_ref[:, :HALF]
    sin = sin_ref[:, :HALF]

    def rope(t, n_heads, out, cosw, sinw):
        for n in range(n_heads):
            t1 = t[:, n * DH:n * DH + HALF]
            t2 = t[:, n * DH + HALF:(n + 1) * DH]
            out[n] = jnp.concatenate([t1 * cosw - t2 * sinw,
                                      t2 * cosw + t1 * sinw],
                                     axis=1).astype(jnp.bfloat16)

    scale = 1.0 / (DH ** 0.5)
    rope(q, H, q_ref, cos * scale, sin * scale)
    rope(k, KVH, k_ref, cos, sin)
    ones = jnp.ones((TS, 1), jnp.bfloat16)
    zeros = jnp.zeros((TS, 128 - DH - 1), jnp.bfloat16)
    for n in range(KVH):
        v_ref[n] = jnp.concatenate(
            [v[:, n * DH:(n + 1) * DH].astype(jnp.bfloat16), ones, zeros],
            axis=1)


def _qkv(x2d, ln1_scale, wq, wk, wv, cos, sin):
    return pl.pallas_call(
        _k1_body,
        grid=(S // TS,),
        in_specs=[
            pl.BlockSpec((TS, D), lambda i: (i, 0)),
            pl.BlockSpec((1, D), lambda i: (0, 0)),
            pl.BlockSpec((D, H * DH), lambda i: (0, 0)),
            pl.BlockSpec((D, KVH * DH), lambda i: (0, 0)),
            pl.BlockSpec((D, KVH * DH), lambda i: (0, 0)),
            pl.BlockSpec((TS, 128), lambda i: (i, 0)),
            pl.BlockSpec((TS, 128), lambda i: (i, 0)),
        ],
        out_specs=[
            pl.BlockSpec((H, TS, DH), lambda i: (0, i, 0)),
            pl.BlockSpec((KVH, TS, DH), lambda i: (0, i, 0)),
            pl.BlockSpec((KVH, TS, 128), lambda i: (0, i, 0)),
        ],
        out_shape=[
            jax.ShapeDtypeStruct((H, S, DH), jnp.bfloat16),
            jax.ShapeDtypeStruct((KVH, S, DH), jnp.bfloat16),
            jax.ShapeDtypeStruct((KVH, S, 128), jnp.bfloat16),
        ],
        compiler_params=pltpu.CompilerParams(
            dimension_semantics=("parallel",)),
    )(x2d, ln1_scale.reshape(1, D), wq, wk, wv, cos, sin)


def _k2_body(iq, q_ref, k_ref, v_ref, o_ref):
    q3 = jnp.concatenate([q_ref[n] for n in range(REP)], axis=0)
    k = k_ref[0]
    s = lax.dot_general(q3, k, (((1,), (1,)), ((), ())),
                        preferred_element_type=jnp.float32)
    tri = (lax.broadcasted_iota(jnp.int32, (REP * TS, TS), 1)
           <= lax.broadcasted_iota(jnp.int32, (REP * TS, TS), 0) % TS)
    diag = jnp.where(tri, s[:, iq * TS:(iq + 1) * TS], -1e30)
    s = diag if iq == 0 else jnp.concatenate([s[:, :iq * TS], diag], axis=1)
    m = jnp.max(s, axis=1, keepdims=True)
    p = jnp.exp(s - m)
    ctx = jnp.dot(p.astype(jnp.bfloat16), v_ref[0],
                  preferred_element_type=jnp.float32)
    l = ctx[:, DH:DH + 1]
    out = (ctx[:, :DH] * (1.0 / l)).astype(jnp.bfloat16)
    for n in range(REP):
        o_ref[n] = out[n * TS:(n + 1) * TS]


def _attn(q, k, v):
    ctxs = []
    for iq in range(S // TS):
        lkv = (iq + 1) * TS
        ctxs.append(pl.pallas_call(
            functools.partial(_k2_body, iq),
            grid=(KVH,),
            in_specs=[
                pl.BlockSpec((REP, TS, DH), lambda g, _i=iq: (g, _i, 0)),
                pl.BlockSpec((1, lkv, DH), lambda g: (g, 0, 0)),
                pl.BlockSpec((1, lkv, 128), lambda g: (g, 0, 0)),
            ],
            out_specs=pl.BlockSpec((REP, TS, DH), lambda g: (g, 0, 0)),
            out_shape=jax.ShapeDtypeStruct((H, TS, DH), jnp.bfloat16),
            compiler_params=pltpu.CompilerParams(
                dimension_semantics=("arbitrary",)),
        )(q, k, v))
    return jnp.concatenate(ctxs, axis=1)


NRB = S // TS


def _top2(lg):
    mx = jnp.max(lg, axis=1, keepdims=True)
    el = jnp.exp(lg - mx)
    probs = el / jnp.sum(el, axis=1, keepdims=True)
    eidx = lax.broadcasted_iota(jnp.int32, (TS, E), 1).astype(jnp.float32)
    m1 = jnp.max(probs, axis=1, keepdims=True)
    idx1 = jnp.min(jnp.where(probs >= m1, eidx, 99.0), axis=1, keepdims=True)
    oh1 = eidx == idx1
    probs2 = jnp.where(oh1, -1.0, probs)
    m2 = jnp.max(probs2, axis=1, keepdims=True)
    idx2 = jnp.min(jnp.where(probs2 >= m2, eidx, 99.0), axis=1, keepdims=True)
    oh2 = eidx == idx2
    ohf = (oh1 | oh2).astype(jnp.float32)
    return m1, oh1, m2, oh2, ohf


def _k3_body(x_ref, ctx_ref, wo_ref, s2_ref, rw_ref,
             x2_ref, h2_ref, lg_ref, cnt_out_ref, cnt_ref):
    i = pl.program_id(0)
    ctx = jnp.concatenate([ctx_ref[n] for n in range(H)], axis=1)
    x2 = x_ref[...] + jnp.dot(ctx, wo_ref[...].astype(jnp.bfloat16),
                              preferred_element_type=jnp.float32)
    var = jnp.mean(x2 * x2, axis=1, keepdims=True)
    h2 = x2 * lax.rsqrt(var + EPS) * s2_ref[...]
    x2_ref[...] = x2
    h2_ref[...] = h2
    lg = jnp.dot(h2, rw_ref[...], preferred_element_type=jnp.float32)
    lg_ref[...] = lg
    _, _, _, _, ohf = _top2(lg)
    cnt_blk = jnp.sum(ohf, axis=0, keepdims=True)

    @pl.when(i == 0)
    def _():
        cnt_ref[...] = jnp.zeros_like(cnt_ref)

    cnt_ref[...] += cnt_blk

    @pl.when(i == NRB - 1)
    def _():
        cnt_out_ref[...] = cnt_ref[...]


def _post_attn(x2d, ctx, wo, ln2_scale, router_w):
    return pl.pallas_call(
        _k3_body,
        grid=(S // TS,),
        in_specs=[
            pl.BlockSpec((TS, D), lambda i: (i, 0)),
            pl.BlockSpec((H, TS, DH), lambda i: (0, i, 0)),
            pl.BlockSpec((H * DH, D), lambda i: (0, 0)),
            pl.BlockSpec((1, D), lambda i: (0, 0)),
            pl.BlockSpec((D, E), lambda i: (0, 0)),
        ],
        out_specs=[
            pl.BlockSpec((TS, D), lambda i: (i, 0)),
            pl.BlockSpec((TS, D), lambda i: (i, 0)),
            pl.BlockSpec((TS, E), lambda i: (i, 0)),
            pl.BlockSpec((1, E), lambda i: (0, 0)),
        ],
        out_shape=[
            jax.ShapeDtypeStruct((S, D), jnp.float32),
            jax.ShapeDtypeStruct((S, D), jnp.float32),
            jax.ShapeDtypeStruct((S, E), jnp.float32),
            jax.ShapeDtypeStruct((1, E), jnp.float32),
        ],
        scratch_shapes=[pltpu.VMEM((1, E), jnp.float32)],
        compiler_params=pltpu.CompilerParams(
            dimension_semantics=("arbitrary",)),
    )(x2d, ctx, wo, ln2_scale.reshape(1, D), router_w)


def _k4_body(lg_ref, cnt_in_ref, pw_ref, be_ref, w0_ref, w1_ref, run_ref):
    j = pl.program_id(0)
    lg = lg_ref[...]
    m1, oh1, m2, oh2, ohf = _top2(lg)
    cnt_blk = jnp.sum(ohf, axis=0, keepdims=True)

    @pl.when(j == 0)
    def _():
        run_ref[...] = jnp.zeros_like(run_ref)

    if True:
        ce = cnt_in_ref[...]
        ce_pad = jnp.ceil(ce * (1.0 / TM)) * TM
        tri = (lax.broadcasted_iota(jnp.int32, (E, E), 0)
               < lax.broadcasted_iota(jnp.int32, (E, E), 1)).astype(jnp.float32)
        offs = jnp.dot(ce_pad, tri, preferred_element_type=jnp.float32)
        lo = (lax.broadcasted_iota(jnp.int32, (TS, TS), 1)
              <= lax.broadcasted_iota(jnp.int32, (TS, TS), 0)).astype(jnp.float32)
        cum = jnp.dot(lo, ohf, preferred_element_type=jnp.float32)
        rank = cum - ohf + run_ref[...]
        pos = offs + rank
        pos0 = jnp.sum(jnp.where(oh1, pos, 0.0), axis=1, keepdims=True)
        pos1 = jnp.sum(jnp.where(oh2, pos, 0.0), axis=1, keepdims=True)
        tot = m1 + m2
        zeros = jnp.zeros((TS, E - 4), jnp.float32)
        pw_ref[...] = jnp.concatenate(
            [pos0, pos1, m1 / tot, m2 / tot, zeros], axis=1)
        w0_ref[...] = jnp.broadcast_to(m1 / tot, (TS, 128))
        w1_ref[...] = jnp.broadcast_to(m2 / tot, (TS, 128))
        run_ref[...] += cnt_blk

        @pl.when(j == NRB - 1)
        def _():
            offs_incl = offs + ce_pad
            nb = (lax.broadcasted_iota(jnp.int32, (128, E), 0)
                  .astype(jnp.float32) * TM)
            cnt_le = jnp.sum((nb >= offs_incl).astype(jnp.float32),
                             axis=1, keepdims=True)
            be = jnp.minimum(cnt_le, float(E - 1))
            tot_pad = offs_incl[0:1, E - 1:E]
            valid = (nb[:, 0:1] < tot_pad).astype(jnp.float32)
            be_ref[...] = jnp.concatenate(
                [be, valid, jnp.zeros((128, E - 2), jnp.float32)], axis=1)


def _route(logits, counts):
    return pl.pallas_call(
        _k4_body,
        grid=(NRB,),
        in_specs=[
            pl.BlockSpec((TS, E), lambda j: (j, 0)),
            pl.BlockSpec((1, E), lambda j: (0, 0)),
        ],
        out_specs=[
            pl.BlockSpec((TS, E), lambda j: (j, 0)),
            pl.BlockSpec((128, E), lambda j: (0, 0)),
            pl.BlockSpec((TS, 128), lambda j: (j, 0)),
            pl.BlockSpec((TS, 128), lambda j: (j, 0)),
        ],
        out_shape=[
            jax.ShapeDtypeStruct((S, E), jnp.float32),
            jax.ShapeDtypeStruct((128, E), jnp.float32),
            jax.ShapeDtypeStruct((S, 128), jnp.float32),
            jax.ShapeDtypeStruct((S, 128), jnp.float32),
        ],
        scratch_shapes=[
            pltpu.VMEM((1, E), jnp.float32),
        ],
        compiler_params=pltpu.CompilerParams(
            dimension_semantics=("arbitrary",)),
    )(logits, counts)


def _dispatch(h2, pos0, pos1, w0, w1):
    mesh = plsc.VectorSubcoreMesh(core_axis_name="c", subcore_axis_name="s")

    @functools.partial(
        pl.kernel,
        out_type=[
            jax.ShapeDtypeStruct((P_TOT, D), jnp.float32),
            jax.ShapeDtypeStruct((P_TOT, 128), jnp.float32),
        ],
        mesh=mesh,
        scratch_types=[
            pltpu.VMEM((CPW,), jnp.int32),
            pltpu.VMEM((CPW,), jnp.int32),
            pltpu.VMEM((CPW, D), jnp.float32),
            pltpu.VMEM((CPW, 128), jnp.float32),
            pltpu.VMEM((CPW, 128), jnp.float32),
            pltpu.SemaphoreType.DMA,
        ],
    )
    def k(h2_hbm, p0_hbm, p1_hbm, w0_hbm, w1_hbm, xs_hbm, ws_hbm,
          idx0_v, idx1_v, rows_v, wv0_v, wv1_v, sem):
        wid = lax.axis_index("s") * 2 + lax.axis_index("c")
        base = wid * CPW
        loads = [
            pltpu.async_copy(p0_hbm.at[pl.ds(base, CPW)], idx0_v, sem),
            pltpu.async_copy(p1_hbm.at[pl.ds(base, CPW)], idx1_v, sem),
            pltpu.async_copy(h2_hbm.at[pl.ds(base, CPW)], rows_v, sem),
            pltpu.async_copy(w0_hbm.at[pl.ds(base, CPW)], wv0_v, sem),
            pltpu.async_copy(w1_hbm.at[pl.ds(base, CPW)], wv1_v, sem),
        ]
        for c in loads:
            c.wait()
        stores = [
            pltpu.async_copy(rows_v, xs_hbm.at[idx0_v], sem),
            pltpu.async_copy(rows_v, xs_hbm.at[idx1_v], sem),
            pltpu.async_copy(wv0_v, ws_hbm.at[idx0_v], sem),
            pltpu.async_copy(wv1_v, ws_hbm.at[idx1_v], sem),
        ]
        for c in stores:
            c.wait()

    return k(h2, pos0, pos1, w0, w1)


def _k6_body(bv_ref, x_ref, wg_ref, wu_ref, wd_ref, ws_ref, o_ref):
    nb = pl.program_id(0)
    nf = pl.program_id(1)

    @pl.when(bv_ref[nb, 1] == 1)
    def _():
        xb = x_ref[...].astype(jnp.bfloat16)
        g = jnp.dot(xb, wg_ref[0].astype(jnp.bfloat16),
                    preferred_element_type=jnp.float32)
        u = jnp.dot(xb, wu_ref[0].astype(jnp.bfloat16),
                    preferred_element_type=jnp.float32)
        a = (g * jax.nn.sigmoid(g) * u).astype(jnp.bfloat16)
        part = jnp.dot(a, wd_ref[0].astype(jnp.bfloat16),
                       preferred_element_type=jnp.float32)
        part = part * ws_ref[:, 0:1]

        @pl.when(nf == 0)
        def _():
            o_ref[...] = part

        @pl.when(nf != 0)
        def _():
            o_ref[...] += part


def _expert_ffn(bv, xs, ws, w_gate, w_up, w_down):
    grid_spec = pltpu.PrefetchScalarGridSpec(
        num_scalar_prefetch=1,
        grid=(NB, NF),
        in_specs=[
            pl.BlockSpec((TM, D), lambda nb, nf, bv: (nb, 0)),
            pl.BlockSpec((1, D, FB), lambda nb, nf, bv: (bv[nb, 0], 0, nf)),
            pl.BlockSpec((1, D, FB), lambda nb, nf, bv: (bv[nb, 0], 0, nf)),
            pl.BlockSpec((1, FB, D), lambda nb, nf, bv: (bv[nb, 0], nf, 0)),
            pl.BlockSpec((TM, 128), lambda nb, nf, bv: (nb, 0)),
        ],
        out_specs=pl.BlockSpec((TM, D), lambda nb, nf, bv: (nb, 0)),
    )
    return pl.pallas_call(
        _k6_body,
        grid_spec=grid_spec,
        out_shape=jax.ShapeDtypeStruct((P_TOT, D), jnp.float32),
        compiler_params=pltpu.CompilerParams(
            dimension_semantics=("arbitrary", "arbitrary")),
    )(bv, xs, w_gate, w_up, w_down, ws)


def _combine(osort, pos0, pos1, x2):
    mesh = plsc.VectorSubcoreMesh(core_axis_name="c", subcore_axis_name="s")

    @functools.partial(
        pl.kernel,
        out_type=jax.ShapeDtypeStruct((S, D), jnp.float32),
        mesh=mesh,
        scratch_types=[
            pltpu.VMEM((CPW,), jnp.int32),
            pltpu.VMEM((CPW,), jnp.int32),
            pltpu.VMEM((CSUB, D), jnp.float32),
            pltpu.VMEM((CSUB, D), jnp.float32),
            pltpu.VMEM((CSUB, D), jnp.float32),
            pltpu.SemaphoreType.DMA,
        ],
    )
    def k(os_hbm, p0_hbm, p1_hbm, x2_hbm, y_hbm,
          idx0_v, idx1_v, r0_v, r1_v, acc_v, sem):
        wid = lax.axis_index("s") * 2 + lax.axis_index("c")
        base = wid * CPW
        pltpu.sync_copy(p0_hbm.at[pl.ds(base, CPW)], idx0_v)
        pltpu.sync_copy(p1_hbm.at[pl.ds(base, CPW)], idx1_v)
        for sub in range(CPW // CSUB):
            row0 = base + sub * CSUB
            cps = [
                pltpu.async_copy(os_hbm.at[idx0_v.at[pl.ds(sub * CSUB, CSUB)]],
                                 r0_v, sem),
                pltpu.async_copy(os_hbm.at[idx1_v.at[pl.ds(sub * CSUB, CSUB)]],
                                 r1_v, sem),
                pltpu.async_copy(x2_hbm.at[pl.ds(row0, CSUB)], acc_v, sem),
            ]
            for c in cps:
                c.wait()

            def token_body(t, _):
                for j in range(D // LANES):
                    sl = pl.ds(j * LANES, LANES)
                    acc_v[t, sl] = acc_v[t, sl] + r0_v[t, sl] + r1_v[t, sl]
                return 0

            lax.fori_loop(0, CSUB, token_body, 0)
            pltpu.sync_copy(acc_v, y_hbm.at[pl.ds(row0, CSUB)])

    return k(osort, pos0, pos1, x2)


def kernel(x, ln1_scale, ln2_scale, wq, wk, wv, wo, router_w, w_gate, w_up, w_down):
    x2d = x.reshape(S, D)
    inv = 1.0 / (10000.0 ** (jnp.arange(0, HALF, dtype=jnp.float32) / HALF))
    ang = jnp.arange(S, dtype=jnp.float32)[:, None] * inv[None, :]
    cos = jnp.concatenate([jnp.cos(ang)] * 4, axis=1)
    sin = jnp.concatenate([jnp.sin(ang)] * 4, axis=1)

    q, k, v = _qkv(x2d, ln1_scale, wq, wk, wv, cos, sin)
    ctx = _attn(q, k, v)
    x2, h2, logits, counts = _post_attn(x2d, ctx, wo, ln2_scale, router_w)
    pw, be_raw, w0, w1 = _route(logits, counts)

    pos0 = pw[:, 0].astype(jnp.int32)
    pos1 = pw[:, 1].astype(jnp.int32)
    bv = be_raw[:NB, :2].astype(jnp.int32)

    xs, ws = _dispatch(h2, pos0, pos1, w0, w1)
    osort = _expert_ffn(bv, xs, ws, w_gate, w_up, w_down)
    y = _combine(osort, pos0, pos1, x2)
    return y.reshape(B, S, D)


# speedup vs baseline: 1.0171x; 1.0171x over previous
import functools

import jax
import jax.numpy as jnp
from jax import lax
from jax.experimental import pallas as pl
from jax.experimental.pallas import tpu as pltpu
from jax.experimental.pallas import tpu_sc as plsc

B, S, D = 1, 2048, 768
H, KVH, DH = 12, 4, 64
E, TOPK, F = 8, 2, 1024
EPS = 1e-6
HALF = DH // 2
REP = H // KVH

TS = 256
TM = 256
NB = (TOPK * S + E * (TM - 1) + TM - 1) // TM
P_TOT = NB * TM
FB = 512
NF = F // FB

NW = 32
CPW = S // NW
CSUB = 16
LANES = 16


def _k1_body(x_ref, s1_ref, wq_ref, wk_ref, wv_ref, cos_ref, sin_ref,
             q_ref, k_ref, v_ref):
    xb = x_ref[...]
    var = jnp.mean(xb * xb, axis=1, keepdims=True)
    h = (xb * lax.rsqrt(var + EPS) * s1_ref[...]).astype(jnp.bfloat16)
    q = jnp.dot(h, wq_ref[...].astype(jnp.bfloat16),
                preferred_element_type=jnp.float32)
    k = jnp.dot(h, wk_ref[...].astype(jnp.bfloat16),
                preferred_element_type=jnp.float32)
    v = jnp.dot(h, wv_ref[...].astype(jnp.bfloat16),
                preferred_element_type=jnp.float32)
    cos = cos_ref[:, :HALF]
    sin = sin_ref[:, :HALF]

    def rope(t, n_heads, out, cosw, sinw):
        for n in range(n_heads):
            t1 = t[:, n * DH:n * DH + HALF]
            t2 = t[:, n * DH + HALF:(n + 1) * DH]
            out[n] = jnp.concatenate([t1 * cosw - t2 * sinw,
                                      t2 * cosw + t1 * sinw],
                                     axis=1).astype(jnp.bfloat16)

    scale = 1.0 / (DH ** 0.5)
    rope(q, H, q_ref, cos * scale, sin * scale)
    rope(k, KVH, k_ref, cos, sin)
    ones = jnp.ones((TS, 1), jnp.bfloat16)
    zeros = jnp.zeros((TS, 128 - DH - 1), jnp.bfloat16)
    for n in range(KVH):
        v_ref[n] = jnp.concatenate(
            [v[:, n * DH:(n + 1) * DH].astype(jnp.bfloat16), ones, zeros],
            axis=1)


def _qkv(x2d, ln1_scale, wq, wk, wv, cos, sin):
    return pl.pallas_call(
        _k1_body,
        grid=(S // TS,),
        in_specs=[
            pl.BlockSpec((TS, D), lambda i: (i, 0)),
            pl.BlockSpec((1, D), lambda i: (0, 0)),
            pl.BlockSpec((D, H * DH), lambda i: (0, 0)),
            pl.BlockSpec((D, KVH * DH), lambda i: (0, 0)),
            pl.BlockSpec((D, KVH * DH), lambda i: (0, 0)),
            pl.BlockSpec((TS, 128), lambda i: (i, 0)),
            pl.BlockSpec((TS, 128), lambda i: (i, 0)),
        ],
        out_specs=[
            pl.BlockSpec((H, TS, DH), lambda i: (0, i, 0)),
            pl.BlockSpec((KVH, TS, DH), lambda i: (0, i, 0)),
            pl.BlockSpec((KVH, TS, 128), lambda i: (0, i, 0)),
        ],
        out_shape=[
            jax.ShapeDtypeStruct((H, S, DH), jnp.bfloat16),
            jax.ShapeDtypeStruct((KVH, S, DH), jnp.bfloat16),
            jax.ShapeDtypeStruct((KVH, S, 128), jnp.bfloat16),
        ],
        compiler_params=pltpu.CompilerParams(
            dimension_semantics=("parallel",)),
    )(x2d, ln1_scale.reshape(1, D), wq, wk, wv, cos, sin)


def _k2_body(iq, q_ref, k_ref, v_ref, o_ref):
    q3 = jnp.concatenate([q_ref[n] for n in range(REP)], axis=0)
    k = k_ref[0]
    s = lax.dot_general(q3, k, (((1,), (1,)), ((), ())),
                        preferred_element_type=jnp.float32)
    tri = (lax.broadcasted_iota(jnp.int32, (REP * TS, TS), 1)
           <= lax.broadcasted_iota(jnp.int32, (REP * TS, TS), 0) % TS)
    diag = jnp.where(tri, s[:, iq * TS:(iq + 1) * TS], -1e30)
    s = diag if iq == 0 else jnp.concatenate([s[:, :iq * TS], diag], axis=1)
    m = jnp.max(s, axis=1, keepdims=True)
    p = jnp.exp(s - m)
    ctx = jnp.dot(p.astype(jnp.bfloat16), v_ref[0],
                  preferred_element_type=jnp.float32)
    l = ctx[:, DH:DH + 1]
    out = (ctx[:, :DH] * (1.0 / l)).astype(jnp.bfloat16)
    for n in range(REP):
        o_ref[n] = out[n * TS:(n + 1) * TS]


def _attn(q, k, v):
    ctxs = []
    for iq in range(S // TS):
        lkv = (iq + 1) * TS
        ctxs.append(pl.pallas_call(
            functools.partial(_k2_body, iq),
            grid=(KVH,),
            in_specs=[
                pl.BlockSpec((REP, TS, DH), lambda g, _i=iq: (g, _i, 0)),
                pl.BlockSpec((1, lkv, DH), lambda g: (g, 0, 0)),
                pl.BlockSpec((1, lkv, 128), lambda g: (g, 0, 0)),
            ],
            out_specs=pl.BlockSpec((REP, TS, DH), lambda g: (g, 0, 0)),
            out_shape=jax.ShapeDtypeStruct((H, TS, DH), jnp.bfloat16),
            compiler_params=pltpu.CompilerParams(
                dimension_semantics=("arbitrary",)),
        )(q, k, v))
    return jnp.concatenate(ctxs, axis=1)


NRB = S // TS


def _top2(lg):
    mx = jnp.max(lg, axis=1, keepdims=True)
    el = jnp.exp(lg - mx)
    probs = el / jnp.sum(el, axis=1, keepdims=True)
    eidx = lax.broadcasted_iota(jnp.int32, (TS, E), 1).astype(jnp.float32)
    m1 = jnp.max(probs, axis=1, keepdims=True)
    idx1 = jnp.min(jnp.where(probs >= m1, eidx, 99.0), axis=1, keepdims=True)
    oh1 = eidx == idx1
    probs2 = jnp.where(oh1, -1.0, probs)
    m2 = jnp.max(probs2, axis=1, keepdims=True)
    idx2 = jnp.min(jnp.where(probs2 >= m2, eidx, 99.0), axis=1, keepdims=True)
    oh2 = eidx == idx2
    ohf = (oh1 | oh2).astype(jnp.float32)
    return m1, oh1, m2, oh2, ohf


def _k3_body(x_ref, ctx_ref, wo_ref, s2_ref, rw_ref,
             x2_ref, h2_ref, lg_ref, cnt_out_ref, cnt_ref):
    i = pl.program_id(0)
    ctx = jnp.concatenate([ctx_ref[n] for n in range(H)], axis=1)
    x2 = x_ref[...] + jnp.dot(ctx, wo_ref[...].astype(jnp.bfloat16),
                              preferred_element_type=jnp.float32)
    var = jnp.mean(x2 * x2, axis=1, keepdims=True)
    h2 = x2 * lax.rsqrt(var + EPS) * s2_ref[...]
    x2_ref[...] = x2
    h2_ref[...] = h2
    lg = jnp.dot(h2, rw_ref[...], preferred_element_type=jnp.float32)
    lg_ref[...] = lg
    _, _, _, _, ohf = _top2(lg)
    cnt_blk = jnp.sum(ohf, axis=0, keepdims=True)

    @pl.when(i == 0)
    def _():
        cnt_ref[...] = jnp.zeros_like(cnt_ref)

    cnt_ref[...] += cnt_blk

    @pl.when(i == NRB - 1)
    def _():
        cnt_out_ref[...] = cnt_ref[...]


def _post_attn(x2d, ctx, wo, ln2_scale, router_w):
    return pl.pallas_call(
        _k3_body,
        grid=(S // TS,),
        in_specs=[
            pl.BlockSpec((TS, D), lambda i: (i, 0)),
            pl.BlockSpec((H, TS, DH), lambda i: (0, i, 0)),
            pl.BlockSpec((H * DH, D), lambda i: (0, 0)),
            pl.BlockSpec((1, D), lambda i: (0, 0)),
            pl.BlockSpec((D, E), lambda i: (0, 0)),
        ],
        out_specs=[
            pl.BlockSpec((TS, D), lambda i: (i, 0)),
            pl.BlockSpec((TS, D), lambda i: (i, 0)),
            pl.BlockSpec((TS, E), lambda i: (i, 0)),
            pl.BlockSpec((1, E), lambda i: (0, 0)),
        ],
        out_shape=[
            jax.ShapeDtypeStruct((S, D), jnp.float32),
            jax.ShapeDtypeStruct((S, D), jnp.float32),
            jax.ShapeDtypeStruct((S, E), jnp.float32),
            jax.ShapeDtypeStruct((1, E), jnp.float32),
        ],
        scratch_shapes=[pltpu.VMEM((1, E), jnp.float32)],
        compiler_params=pltpu.CompilerParams(
            dimension_semantics=("arbitrary",)),
    )(x2d, ctx, wo, ln2_scale.reshape(1, D), router_w)


def _k4_body(lg_ref, cnt_in_ref, pw_ref, be_ref, w0_ref, w1_ref, run_ref):
    j = pl.program_id(0)
    lg = lg_ref[...]
    m1, oh1, m2, oh2, ohf = _top2(lg)
    cnt_blk = jnp.sum(ohf, axis=0, keepdims=True)

    @pl.when(j == 0)
    def _():
        run_ref[...] = jnp.zeros_like(run_ref)

    if True:
        ce = cnt_in_ref[...]
        ce_pad = jnp.ceil(ce * (1.0 / TM)) * TM
        tri = (lax.broadcasted_iota(jnp.int32, (E, E), 0)
               < lax.broadcasted_iota(jnp.int32, (E, E), 1)).astype(jnp.float32)
        offs = jnp.dot(ce_pad, tri, preferred_element_type=jnp.float32)
        lo = (lax.broadcasted_iota(jnp.int32, (TS, TS), 1)
              <= lax.broadcasted_iota(jnp.int32, (TS, TS), 0)).astype(jnp.float32)
        cum = jnp.dot(lo, ohf, preferred_element_type=jnp.float32)
        rank = cum - ohf + run_ref[...]
        pos = offs + rank
        pos0 = jnp.sum(jnp.where(oh1, pos, 0.0), axis=1, keepdims=True)
        pos1 = jnp.sum(jnp.where(oh2, pos, 0.0), axis=1, keepdims=True)
        tot = m1 + m2
        zeros = jnp.zeros((TS, E - 4), jnp.float32)
        pw_ref[...] = jnp.concatenate(
            [pos0, pos1, m1 / tot, m2 / tot, zeros], axis=1)
        w0_ref[...] = jnp.broadcast_to(m1 / tot, (TS, 128))
        w1_ref[...] = jnp.broadcast_to(m2 / tot, (TS, 128))
        run_ref[...] += cnt_blk

        @pl.when(j == NRB - 1)
        def _():
            offs_incl = offs + ce_pad
            nb = (lax.broadcasted_iota(jnp.int32, (128, E), 0)
                  .astype(jnp.float32) * TM)
            cnt_le = jnp.sum((nb >= offs_incl).astype(jnp.float32),
                             axis=1, keepdims=True)
            be = jnp.minimum(cnt_le, float(E - 1))
            tot_pad = offs_incl[0:1, E - 1:E]
            valid = (nb[:, 0:1] < tot_pad).astype(jnp.float32)
            be_ref[...] = jnp.concatenate(
                [be, valid, jnp.zeros((128, E - 2), jnp.float32)], axis=1)


def _route(logits, counts):
    return pl.pallas_call(
        _k4_body,
        grid=(NRB,),
        in_specs=[
            pl.BlockSpec((TS, E), lambda j: (j, 0)),
            pl.BlockSpec((1, E), lambda j: (0, 0)),
        ],
        out_specs=[
            pl.BlockSpec((TS, E), lambda j: (j, 0)),
            pl.BlockSpec((128, E), lambda j: (0, 0)),
            pl.BlockSpec((TS, 128), lambda j: (j, 0)),
            pl.BlockSpec((TS, 128), lambda j: (j, 0)),
        ],
        out_shape=[
            jax.ShapeDtypeStruct((S, E), jnp.float32),
            jax.ShapeDtypeStruct((128, E), jnp.float32),
            jax.ShapeDtypeStruct((S, 128), jnp.float32),
            jax.ShapeDtypeStruct((S, 128), jnp.float32),
        ],
        scratch_shapes=[
            pltpu.VMEM((1, E), jnp.float32),
        ],
        compiler_params=pltpu.CompilerParams(
            dimension_semantics=("arbitrary",)),
    )(logits, counts)


def _dispatch(h2, pos0, pos1, w0, w1):
    mesh = plsc.VectorSubcoreMesh(core_axis_name="c", subcore_axis_name="s")

    @functools.partial(
        pl.kernel,
        out_type=[
            jax.ShapeDtypeStruct((P_TOT, D), jnp.float32),
            jax.ShapeDtypeStruct((P_TOT, 128), jnp.float32),
        ],
        mesh=mesh,
        scratch_types=[
            pltpu.VMEM((CPW,), jnp.int32),
            pltpu.VMEM((CPW,), jnp.int32),
            pltpu.VMEM((CPW, D), jnp.float32),
            pltpu.VMEM((CPW, 128), jnp.float32),
            pltpu.VMEM((CPW, 128), jnp.float32),
            pltpu.SemaphoreType.DMA,
        ],
    )
    def k(h2_hbm, p0_hbm, p1_hbm, w0_hbm, w1_hbm, xs_hbm, ws_hbm,
          idx0_v, idx1_v, rows_v, wv0_v, wv1_v, sem):
        wid = lax.axis_index("s") * 2 + lax.axis_index("c")
        base = wid * CPW
        loads = [
            pltpu.async_copy(p0_hbm.at[pl.ds(base, CPW)], idx0_v, sem),
            pltpu.async_copy(p1_hbm.at[pl.ds(base, CPW)], idx1_v, sem),
            pltpu.async_copy(h2_hbm.at[pl.ds(base, CPW)], rows_v, sem),
            pltpu.async_copy(w0_hbm.at[pl.ds(base, CPW)], wv0_v, sem),
            pltpu.async_copy(w1_hbm.at[pl.ds(base, CPW)], wv1_v, sem),
        ]
        for c in loads:
            c.wait()
        stores = [
            pltpu.async_copy(rows_v, xs_hbm.at[idx0_v], sem),
            pltpu.async_copy(rows_v, xs_hbm.at[idx1_v], sem),
            pltpu.async_copy(wv0_v, ws_hbm.at[idx0_v], sem),
            pltpu.async_copy(wv1_v, ws_hbm.at[idx1_v], sem),
        ]
        for c in stores:
            c.wait()

    return k(h2, pos0, pos1, w0, w1)


def _k6_body(bv_ref, x_ref, wg_ref, wu_ref, wd_ref, ws_ref, o_ref):
    nb = pl.program_id(0)
    nf = pl.program_id(1)

    @pl.when(bv_ref[nb, 1] == 1)
    def _():
        xb = x_ref[...].astype(jnp.bfloat16)
        g = jnp.dot(xb, wg_ref[0].astype(jnp.bfloat16),
                    preferred_element_type=jnp.float32)
        u = jnp.dot(xb, wu_ref[0].astype(jnp.bfloat16),
                    preferred_element_type=jnp.float32)
        a = (g * jax.nn.sigmoid(g) * u).astype(jnp.bfloat16)
        part = jnp.dot(a, wd_ref[0].astype(jnp.bfloat16),
                       preferred_element_type=jnp.float32)
        part = part * ws_ref[:, 0:1]

        @pl.when(nf == 0)
        def _():
            o_ref[...] = part

        @pl.when(nf != 0)
        def _():
            o_ref[...] += part


def _expert_ffn(bv, xs, ws, w_gate, w_up, w_down):
    grid_spec = pltpu.PrefetchScalarGridSpec(
        num_scalar_prefetch=1,
        grid=(NB, NF),
        in_specs=[
            pl.BlockSpec((TM, D), lambda nb, nf, bv: (nb, 0)),
            pl.BlockSpec((1, D, FB), lambda nb, nf, bv: (bv[nb, 0], 0, nf)),
            pl.BlockSpec((1, D, FB), lambda nb, nf, bv: (bv[nb, 0], 0, nf)),
            pl.BlockSpec((1, FB, D), lambda nb, nf, bv: (bv[nb, 0], nf, 0)),
            pl.BlockSpec((TM, 128), lambda nb, nf, bv: (nb, 0)),
        ],
        out_specs=pl.BlockSpec((TM, D), lambda nb, nf, bv: (nb, 0)),
    )
    return pl.pallas_call(
        _k6_body,
        grid_spec=grid_spec,
        out_shape=jax.ShapeDtypeStruct((P_TOT, D), jnp.float32),
        compiler_params=pltpu.CompilerParams(
            dimension_semantics=("arbitrary", "arbitrary")),
    )(bv, xs, w_gate, w_up, w_down, ws)


def _combine(osort, pos0, pos1, x2):
    mesh = plsc.VectorSubcoreMesh(core_axis_name="c", subcore_axis_name="s")

    @functools.partial(
        pl.kernel,
        out_type=jax.ShapeDtypeStruct((S, D), jnp.float32),
        mesh=mesh,
        scratch_types=[
            pltpu.VMEM((CPW,), jnp.int32),
            pltpu.VMEM((CPW,), jnp.int32),
            pltpu.VMEM((CSUB, D), jnp.float32),
            pltpu.VMEM((CSUB, D), jnp.float32),
            pltpu.VMEM((CSUB, D), jnp.float32),
            pltpu.VMEM((CSUB, D), jnp.float32),
            pltpu.VMEM((CSUB, D), jnp.float32),
            pltpu.VMEM((CSUB, D), jnp.float32),
            pltpu.SemaphoreType.DMA,
            pltpu.SemaphoreType.DMA,
        ],
    )
    def k(os_hbm, p0_hbm, p1_hbm, x2_hbm, y_hbm,
          idx0_v, idx1_v, r0_a, r1_a, acc_a, r0_b, r1_b, acc_b, sem_a, sem_b):
        wid = lax.axis_index("s") * 2 + lax.axis_index("c")
        base = wid * CPW
        nsub = CPW // CSUB
        pltpu.sync_copy(p0_hbm.at[pl.ds(base, CPW)], idx0_v)
        pltpu.sync_copy(p1_hbm.at[pl.ds(base, CPW)], idx1_v)
        bufs = ((r0_a, r1_a, acc_a, sem_a), (r0_b, r1_b, acc_b, sem_b))

        def fire(sub):
            r0, r1, acc, sem = bufs[sub % 2]
            return [
                pltpu.async_copy(os_hbm.at[idx0_v.at[pl.ds(sub * CSUB, CSUB)]],
                                 r0, sem),
                pltpu.async_copy(os_hbm.at[idx1_v.at[pl.ds(sub * CSUB, CSUB)]],
                                 r1, sem),
                pltpu.async_copy(x2_hbm.at[pl.ds(base + sub * CSUB, CSUB)],
                                 acc, sem),
            ]

        pend = fire(0)
        for sub in range(nsub):
            for c in pend:
                c.wait()
            if sub + 1 < nsub:
                pend = fire(sub + 1)
            r0, r1, acc, _ = bufs[sub % 2]

            def token_body(t, _, r0=r0, r1=r1, acc=acc):
                for j in range(D // LANES):
                    sl = pl.ds(j * LANES, LANES)
                    acc[t, sl] = acc[t, sl] + r0[t, sl] + r1[t, sl]
                return 0

            lax.fori_loop(0, CSUB, token_body, 0)
            pltpu.sync_copy(acc, y_hbm.at[pl.ds(base + sub * CSUB, CSUB)])

    return k(osort, pos0, pos1, x2)


def kernel(x, ln1_scale, ln2_scale, wq, wk, wv, wo, router_w, w_gate, w_up, w_down):
    x2d = x.reshape(S, D)
    inv = 1.0 / (10000.0 ** (jnp.arange(0, HALF, dtype=jnp.float32) / HALF))
    ang = jnp.arange(S, dtype=jnp.float32)[:, None] * inv[None, :]
    cos = jnp.concatenate([jnp.cos(ang)] * 4, axis=1)
    sin = jnp.concatenate([jnp.sin(ang)] * 4, axis=1)

    q, k, v = _qkv(x2d, ln1_scale, wq, wk, wv, cos, sin)
    ctx = _attn(q, k, v)
    x2, h2, logits, counts = _post_attn(x2d, ctx, wo, ln2_scale, router_w)
    pw, be_raw, w0, w1 = _route(logits, counts)

    pos0 = pw[:, 0].astype(jnp.int32)
    pos1 = pw[:, 1].astype(jnp.int32)
    bv = be_raw[:NB, :2].astype(jnp.int32)

    xs, ws = _dispatch(h2, pos0, pos1, w0, w1)
    osort = _expert_ffn(bv, xs, ws, w_gate, w_up, w_down)
    y = _combine(osort, pos0, pos1, x2)
    return y.reshape(B, S, D)
